# Initial kernel scaffold; baseline (speedup 1.0000x reference)
import jax
import jax.numpy as jnp
from jax.experimental import pallas as pl


def kernel(dist_softmax, W1, b1, W2, b2):
    raise NotImplementedError("write your pallas kernel here")


# speedup vs baseline: 1.7257x; 1.7257x over previous
import functools

import jax
import jax.numpy as jnp
from jax import lax
from jax.experimental import pallas as pl
from jax.experimental.pallas import tpu as pltpu
from jax.experimental.pallas import tpu_sc as plsc

B = 16
NQ = 20000
N = B * NQ
G = 4
E = 17
ROW = G * E
K = 4

NC, NS, L = 2, 16, 16
NW = NC * NS
A = 128
GROUPS = A // L
CB = N // A
CB_LO, CB_XTRA = divmod(CB, NW)

_sc_mesh = plsc.VectorSubcoreMesh(
    core_axis_name="c", subcore_axis_name="s", num_cores=NC, num_subcores=NS
)


def _insert(ms, v):
  out = []
  cur = v
  full = len(ms) == K
  for i, t in enumerate(ms):
    out.append(jnp.maximum(t, cur))
    if not (full and i == len(ms) - 1):
      cur = jnp.minimum(t, cur)
  if not full:
    out.append(cur)
  return out[:K]


@functools.partial(
    pl.kernel,
    out_type=jax.ShapeDtypeStruct((B, N), jnp.float32),
    mesh=_sc_mesh,
    scratch_types=[
        pltpu.VMEM((A * ROW,), jnp.float32),
        pltpu.VMEM((B, A), jnp.float32),
    ],
    compiler_params=pltpu.CompilerParams(needs_layout_passes=False),
)
def _sc_topk(dist_hbm, stat_hbm, inbuf, outbuf):
  w = lax.axis_index("s") * NC + lax.axis_index("c")
  start = w * CB_LO + jnp.minimum(w, CB_XTRA)
  nblk = CB_LO + jnp.where(w < CB_XTRA, 1, 0)
  iota68 = lax.iota(jnp.int32, L) * ROW

  @pl.loop(0, nblk)
  def _piece(p):
    abase = (start + p) * A
    pltpu.sync_copy(dist_hbm.at[pl.ds(abase * ROW, A * ROW)], inbuf)

    @pl.loop(0, GROUPS)
    def _group(gi):
      a0 = gi * L
      bidx = iota68 + a0 * ROW
      for g in range(G):
        ms = []
        for j in range(E):
          v = plsc.load_gather(inbuf, [bidx + (g * E + j)])
          ms = _insert(ms, v)
        for k in range(K):
          outbuf[g * K + k, pl.ds(a0, L)] = ms[k]

    pltpu.sync_copy(outbuf, stat_hbm.at[:, pl.ds(abase, A)])


BN = 12800


def _tc_body(stat_ref, w1_ref, b1_ref, w2_ref, b2_ref, out_ref):
  x = stat_ref[...]
  h = lax.dot_general(
      w1_ref[...], x, (((1,), (0,)), ((), ())),
      preferred_element_type=jnp.float32,
  )
  h = jnp.maximum(h + b1_ref[...], 0.0)
  y = jnp.sum(h * w2_ref[...], axis=0, keepdims=True) + b2_ref[...]
  out_ref[...] = jax.nn.sigmoid(y).reshape(1, 1, BN)


_tc_mlp = pl.pallas_call(
    _tc_body,
    grid=(N // BN,),
    in_specs=[
        pl.BlockSpec((B, BN), lambda i: (0, i)),
        pl.BlockSpec((64, B), lambda i: (0, 0)),
        pl.BlockSpec((64, 1), lambda i: (0, 0)),
        pl.BlockSpec((64, 1), lambda i: (0, 0)),
        pl.BlockSpec((1, 1), lambda i: (0, 0)),
    ],
    out_specs=pl.BlockSpec((1, 1, BN), lambda i: (i, 0, 0)),
    out_shape=jax.ShapeDtypeStruct((N // BN, 1, BN), jnp.float32),
)


def kernel(dist_softmax, W1, b1, W2, b2):
  dist_flat = dist_softmax.reshape(-1)
  stat_t = _sc_topk(dist_flat)
  w1r = W1.reshape(64, G, K + 1)
  w1e = (w1r[:, :, :K] + 0.25 * w1r[:, :, K:]).reshape(64, B)
  out = _tc_mlp(stat_t, w1e, b1.reshape(64, 1), W2.reshape(64, 1),
                b2.reshape(1, 1))
  return out.reshape(B, NQ)


# speedup vs baseline: 17.7306x; 10.2742x over previous
import functools

import jax
import jax.numpy as jnp
from jax import lax
from jax.experimental import pallas as pl
from jax.experimental.pallas import tpu as pltpu
from jax.experimental.pallas import tpu_sc as plsc

B = 16
NQ = 20000
G = 4
E = 17
K = 4
F = G * K

NC, NS, L = 2, 16, 16
NW = NC * NS

Q = 384
TPB = 19968 // Q
PPW = B * TPB // NW
QT = TPB * Q
QTW = NQ - QT

_sc_mesh = plsc.VectorSubcoreMesh(
    core_axis_name="c", subcore_axis_name="s", num_cores=NC, num_subcores=NS
)


def _insert(ms, v):
  out = []
  cur = v
  full = len(ms) == K
  for i, t in enumerate(ms):
    out.append(jnp.maximum(t, cur))
    if not (full and i == len(ms) - 1):
      cur = jnp.minimum(t, cur)
  if not full:
    out.append(cur)
  return out[:K]


@functools.partial(
    pl.kernel,
    out_type=jax.ShapeDtypeStruct((B, F, NQ), jnp.float32),
    mesh=_sc_mesh,
    scratch_types=[
        pltpu.VMEM((E, G, Q), jnp.float32),
        pltpu.VMEM((E, G, Q), jnp.float32),
        pltpu.VMEM((F, Q), jnp.float32),
        pltpu.VMEM((E, G, QTW), jnp.float32),
        pltpu.VMEM((F, QTW), jnp.float32),
        pltpu.SemaphoreType.DMA,
        pltpu.SemaphoreType.DMA,
    ],
    compiler_params=pltpu.CompilerParams(needs_layout_passes=False),
)
def _sc_topk(dist_hbm, stat_hbm, buf0, buf1, outbuf, tbuf, toutbuf, sem0, sem1):
  w = lax.axis_index("s") * NC + lax.axis_index("c")
  p0g = w * PPW

  def src(p):
    b = p // TPB
    t = p - b * TPB
    return dist_hbm.at[b, :, :, pl.ds(t * Q, Q)]

  def topk_sweep(buf, obuf, ngroups):
    @pl.loop(0, ngroups)
    def _g(qi):
      a0 = qi * L
      for g in range(G):
        ms = []
        for e in range(E):
          ms = _insert(ms, buf[e, g, pl.ds(a0, L)])
        for k in range(K):
          obuf[g * K + k, pl.ds(a0, L)] = ms[k]

  def out_dma(p):
    b = p // TPB
    t = p - b * TPB
    pltpu.sync_copy(outbuf, stat_hbm.at[b, :, pl.ds(t * Q, Q)])

  pltpu.async_copy(src(p0g), buf0, sem0)

  @pl.loop(0, PPW // 2)
  def _body(j):
    pa = p0g + 2 * j
    pb = pa + 1
    pltpu.async_copy(src(pb), buf1, sem1)
    pltpu.make_async_copy(src(pa), buf0, sem0).wait()
    topk_sweep(buf0, outbuf, Q // L)
    out_dma(pa)

    @pl.when(j < PPW // 2 - 1)
    def _():
      pltpu.async_copy(src(pa + 2), buf0, sem0)

    pltpu.make_async_copy(src(pb), buf1, sem1).wait()
    topk_sweep(buf1, outbuf, Q // L)
    out_dma(pb)

  @pl.when(w < B)
  def _tail():
    pltpu.sync_copy(dist_hbm.at[w, :, :, pl.ds(QT, QTW)], tbuf)
    topk_sweep(tbuf, toutbuf, QTW // L)
    pltpu.sync_copy(toutbuf, stat_hbm.at[w, :, pl.ds(QT, QTW)])


def _tc_body(stat_ref, w1_ref, b1_ref, w2_ref, b2_ref, out_ref):
  x = stat_ref[0]
  h = lax.dot_general(
      w1_ref[...], x, (((1,), (0,)), ((), ())),
      preferred_element_type=jnp.float32,
  )
  h = jnp.maximum(h + b1_ref[...], 0.0)
  y = jnp.sum(h * w2_ref[...], axis=0, keepdims=True) + b2_ref[...]
  out_ref[...] = jax.nn.sigmoid(y)[None]


_tc_mlp = pl.pallas_call(
    _tc_body,
    grid=(B,),
    in_specs=[
        pl.BlockSpec((1, F, NQ), lambda i: (i, 0, 0)),
        pl.BlockSpec((64, F), lambda i: (0, 0)),
        pl.BlockSpec((64, 1), lambda i: (0, 0)),
        pl.BlockSpec((64, 1), lambda i: (0, 0)),
        pl.BlockSpec((1, 1), lambda i: (0, 0)),
    ],
    out_specs=pl.BlockSpec((1, 1, NQ), lambda i: (i, 0, 0)),
    out_shape=jax.ShapeDtypeStruct((B, 1, NQ), jnp.float32),
)


def kernel(dist_softmax, W1, b1, W2, b2):
  dist_t = jnp.transpose(dist_softmax, (0, 3, 2, 1))
  stat3 = _sc_topk(dist_t)
  w1r = W1.reshape(64, G, K + 1)
  w1e = (w1r[:, :, :K] + 0.25 * w1r[:, :, K:]).reshape(64, F)
  out3 = _tc_mlp(stat3, w1e, b1.reshape(64, 1), W2.reshape(64, 1),
                 b2.reshape(1, 1))
  return out3.reshape(B, NQ)
